# initial kernel scaffold (unmeasured)
import jax
import jax.numpy as jnp
from jax import lax
from jax.experimental import pallas as pl
from jax.experimental.pallas import tpu as pltpu

N_DEV = 8
M_PER = 4096
CHUNK = M_PER // N_DEV
K = 1024
N = 1024


def kernel(t, W):
    t = t.astype(jnp.bfloat16)
    W = W.astype(jnp.bfloat16)

    def body(t_ref, w_ref, out_ref,
             rs_send, rs_recv, ag_buf,
             rs_ssem, rs_rsem, ag_ssem, ag_rsem):
        me = lax.axis_index("i")
        right = lax.rem(me + 1, N_DEV)
        left = lax.rem(me + N_DEV - 1, N_DEV)

        def mod8(v):
            return lax.rem(v + 2 * N_DEV, N_DEV)

        def t_chunk(idx):
            return t_ref[pl.ds(idx * CHUNK, CHUNK), :]

        barrier_sem = pltpu.get_barrier_semaphore()
        pl.semaphore_signal(barrier_sem, inc=1, device_id=(left,),
                            device_id_type=pl.DeviceIdType.MESH)
        pl.semaphore_signal(barrier_sem, inc=1, device_id=(right,),
                            device_id_type=pl.DeviceIdType.MESH)
        pl.semaphore_wait(barrier_sem, 2)

        rs_send[0] = t_chunk(me)
        for h in range(N_DEV - 1):
            rdma = pltpu.make_async_remote_copy(
                src_ref=rs_send.at[h],
                dst_ref=rs_recv.at[h],
                send_sem=rs_ssem.at[h],
                recv_sem=rs_rsem.at[h],
                device_id=(right,),
                device_id_type=pl.DeviceIdType.MESH,
            )
            rdma.start()
            rdma.wait()
            r_h = mod8(me - h - 1)
            if h < N_DEV - 2:
                rs_send[h + 1] = rs_recv[h] + t_chunk(r_h)

        own = mod8(me + 1)
        own_val = rs_recv[N_DEV - 2] + t_chunk(own)

        result = jnp.dot(own_val, w_ref[:, :],
                         preferred_element_type=jnp.float32)
        out_ref[pl.ds(own * CHUNK, CHUNK), :] = result
        ag_buf[0] = result.astype(jnp.bfloat16)

        for h in range(N_DEV - 1):
            rdma = pltpu.make_async_remote_copy(
                src_ref=ag_buf.at[h],
                dst_ref=ag_buf.at[h + 1],
                send_sem=ag_ssem.at[h],
                recv_sem=ag_rsem.at[h],
                device_id=(right,),
                device_id_type=pl.DeviceIdType.MESH,
            )
            rdma.start()
            rdma.wait()
            origin = mod8(me - h)
            out_ref[pl.ds(origin * CHUNK, CHUNK), :] = (
                ag_buf[h + 1].astype(jnp.float32))

    return pl.pallas_call(
        body,
        out_shape=jax.ShapeDtypeStruct((M_PER, N), jnp.float32),
        in_specs=[
            pl.BlockSpec(memory_space=pltpu.VMEM),
            pl.BlockSpec(memory_space=pltpu.VMEM),
        ],
        out_specs=pl.BlockSpec(memory_space=pltpu.VMEM),
        scratch_shapes=[
            pltpu.VMEM((N_DEV - 1, CHUNK, K), jnp.bfloat16),
            pltpu.VMEM((N_DEV - 1, CHUNK, K), jnp.bfloat16),
            pltpu.VMEM((N_DEV, CHUNK, N), jnp.bfloat16),
            pltpu.SemaphoreType.DMA((N_DEV - 1,)),
            pltpu.SemaphoreType.DMA((N_DEV - 1,)),
            pltpu.SemaphoreType.DMA((N_DEV - 1,)),
            pltpu.SemaphoreType.DMA((N_DEV - 1,)),
        ],
        compiler_params=pltpu.CompilerParams(collective_id=0),
    )(t, W)


# baseline (device time: 225253 ns/iter reference)
import jax
import jax.numpy as jnp
from jax import lax
from jax.experimental import pallas as pl
from jax.experimental.pallas import tpu as pltpu

N_DEV = 8
M_PER = 4096
CHUNK = M_PER // N_DEV
K = 1024
N = 1024


def kernel(t, W):
    t = t.astype(jnp.bfloat16)
    W = W.astype(jnp.bfloat16)

    def body(t_ref, w_ref, out_ref,
             rs_send, rs_recv, ag_buf,
             rs_ssem, rs_rsem, ag_ssem, ag_rsem):
        me = lax.axis_index("i")
        right = lax.rem(me + 1, N_DEV)
        left = lax.rem(me + N_DEV - 1, N_DEV)

        def mod8(v):
            return lax.rem(v + 2 * N_DEV, N_DEV)

        def t_chunk(idx):
            return t_ref[pl.ds(idx * CHUNK, CHUNK), :]

        barrier_sem = pltpu.get_barrier_semaphore()
        pl.semaphore_signal(barrier_sem, inc=1, device_id=(left,),
                            device_id_type=pl.DeviceIdType.MESH)
        pl.semaphore_signal(barrier_sem, inc=1, device_id=(right,),
                            device_id_type=pl.DeviceIdType.MESH)
        pl.semaphore_wait(barrier_sem, 2)

        rs_send[0] = t_chunk(me)
        for h in range(N_DEV - 1):
            rdma = pltpu.make_async_remote_copy(
                src_ref=rs_send.at[h],
                dst_ref=rs_recv.at[h],
                send_sem=rs_ssem.at[h],
                recv_sem=rs_rsem.at[h],
                device_id=(right,),
                device_id_type=pl.DeviceIdType.MESH,
            )
            rdma.start()
            rdma.wait()
            r_h = mod8(me - h - 1)
            if h < N_DEV - 2:
                rs_send[h + 1] = rs_recv[h] + t_chunk(r_h)

        own = mod8(me + 1)
        own_val = rs_recv[N_DEV - 2] + t_chunk(own)

        result = jnp.dot(own_val, w_ref[:, :],
                         preferred_element_type=jnp.float32)
        out_ref[pl.ds(own * CHUNK, CHUNK), :] = result
        ag_buf[0] = result.astype(jnp.bfloat16)

        for h in range(N_DEV - 1):
            rdma = pltpu.make_async_remote_copy(
                src_ref=ag_buf.at[h],
                dst_ref=ag_buf.at[h + 1],
                send_sem=ag_ssem.at[h],
                recv_sem=ag_rsem.at[h],
                device_id=(right,),
                device_id_type=pl.DeviceIdType.MESH,
            )
            rdma.start()
            rdma.wait()
            origin = mod8(me - h)
            out_ref[pl.ds(origin * CHUNK, CHUNK), :] = (
                ag_buf[h + 1].astype(jnp.float32))

    return pl.pallas_call(
        body,
        out_shape=jax.ShapeDtypeStruct((M_PER, N), jnp.float32),
        in_specs=[
            pl.BlockSpec(memory_space=pltpu.VMEM),
            pl.BlockSpec(memory_space=pltpu.VMEM),
        ],
        out_specs=pl.BlockSpec(memory_space=pltpu.VMEM),
        scratch_shapes=[
            pltpu.VMEM((N_DEV - 1, CHUNK, K), jnp.bfloat16),
            pltpu.VMEM((N_DEV - 1, CHUNK, K), jnp.bfloat16),
            pltpu.VMEM((N_DEV, CHUNK, N), jnp.bfloat16),
            pltpu.SemaphoreType.DMA((N_DEV - 1,)),
            pltpu.SemaphoreType.DMA((N_DEV - 1,)),
            pltpu.SemaphoreType.DMA((N_DEV - 1,)),
            pltpu.SemaphoreType.DMA((N_DEV - 1,)),
        ],
        compiler_params=pltpu.CompilerParams(
            collective_id=0,
            vmem_limit_bytes=64 * 1024 * 1024,
        ),
    )(t, W)


# device time: 133760 ns/iter; 1.6840x vs baseline; 1.6840x over previous
import jax
import jax.numpy as jnp
from jax import lax
from jax.experimental import pallas as pl
from jax.experimental.pallas import tpu as pltpu

N_DEV = 8
M_PER = 4096
ROWS = M_PER // N_DEV
K = 1024
N = 1024
HC = K // 2

RS_MASKS = ((1, 3, 4), (3, 4, 1))
AG_MASKS = ((4, 3, 1), (1, 4, 3))


def _sigma(bf: int, u: int) -> int:
    if bf == 0:
        a, b, c = (u >> 2) & 1, (u >> 1) & 1, u & 1
    else:
        b, c, a = (u >> 2) & 1, (u >> 1) & 1, u & 1
    return (a ^ b) | (b << 1) | (c << 2)


def kernel(t, W):
    t = t.astype(jnp.bfloat16)
    W = W.astype(jnp.bfloat16)

    def body(t_ref, w_ref, out_ref, accA, accB,
             rsr0A, rsr0B, rsr1A, rsr1B, rsr2A, rsr2B,
             agr0A, agr0B, agr1A, agr1B, agr2A, agr2B,
             rs_ssem, rs_rsem, ag_ssem, ag_rsem):
        acc = (accA, accB)
        rsr = ((rsr0A, rsr1A, rsr2A), (rsr0B, rsr1B, rsr2B))
        agr = ((agr0A, agr1A, agr2A), (agr0B, agr1B, agr2B))

        me = lax.axis_index("i")
        p0, p1, p2 = me % 2, (me // 2) % 2, (me // 4) % 2
        f1, f3, f4 = p0 ^ p1, p1, p2
        vv = (4 * f1 + 2 * f3 + f4, 4 * f3 + 2 * f4 + f1)

        barrier_sem = pltpu.get_barrier_semaphore()
        for mask in (1, 3, 4):
            pl.semaphore_signal(barrier_sem, inc=1, device_id=(me ^ mask,),
                                device_id_type=pl.DeviceIdType.MESH)
        pl.semaphore_wait(barrier_sem, 3)

        for bf in range(2):
            for u in range(N_DEV):
                acc[bf][pl.ds(u * ROWS, ROWS), :] = t_ref[
                    pl.ds(_sigma(bf, u) * ROWS, ROWS), pl.ds(bf * HC, HC)]

        for s in range(3):
            half = 4 >> s
            waits = []
            for bf in range(2):
                v = vv[bf]
                base = (v // (8 >> s)) * (8 >> s)
                halfbit = (v // half) % 2
                send_base = base + (1 - halfbit) * half
                keep_base = base + halfbit * half
                rdma = pltpu.make_async_remote_copy(
                    src_ref=acc[bf].at[pl.ds(send_base * ROWS, half * ROWS), :],
                    dst_ref=rsr[bf][s],
                    send_sem=rs_ssem.at[2 * s + bf],
                    recv_sem=rs_rsem.at[2 * s + bf],
                    device_id=(me ^ RS_MASKS[bf][s],),
                    device_id_type=pl.DeviceIdType.MESH,
                )
                rdma.start()
                waits.append((rdma, bf, keep_base))
            for rdma, bf, keep_base in waits:
                rdma.wait()
                sl = pl.ds(keep_base * ROWS, half * ROWS)
                acc[bf][sl, :] = acc[bf][sl, :] + rsr[bf][s][:, :]

        own = jnp.concatenate(
            [acc[0][pl.ds(vv[0] * ROWS, ROWS), :],
             acc[1][pl.ds(vv[1] * ROWS, ROWS), :]], axis=1)
        result = jnp.dot(own, w_ref[:, :], preferred_element_type=jnp.float32)
        rb = result.astype(jnp.bfloat16)
        acc[0][pl.ds(vv[0] * ROWS, ROWS), :] = rb[:, :HC]
        acc[1][pl.ds(vv[1] * ROWS, ROWS), :] = rb[:, HC:]

        for s in range(3):
            bsz = 1 << s
            waits = []
            for bf in range(2):
                v = vv[bf]
                b_s = (v // bsz) * bsz
                bit_s = (v // bsz) % 2
                recv_base = b_s + (1 - 2 * bit_s) * bsz
                rdma = pltpu.make_async_remote_copy(
                    src_ref=acc[bf].at[pl.ds(b_s * ROWS, bsz * ROWS), :],
                    dst_ref=agr[bf][s],
                    send_sem=ag_ssem.at[2 * s + bf],
                    recv_sem=ag_rsem.at[2 * s + bf],
                    device_id=(me ^ AG_MASKS[bf][s],),
                    device_id_type=pl.DeviceIdType.MESH,
                )
                rdma.start()
                waits.append((rdma, bf, recv_base))
            for rdma, bf, recv_base in waits:
                rdma.wait()
                acc[bf][pl.ds(recv_base * ROWS, bsz * ROWS), :] = agr[bf][s][:, :]

        for bf in range(2):
            for u in range(N_DEV):
                out_ref[pl.ds(_sigma(bf, u) * ROWS, ROWS),
                        pl.ds(bf * HC, HC)] = (
                    acc[bf][pl.ds(u * ROWS, ROWS), :].astype(jnp.float32))

    bf16 = jnp.bfloat16
    return pl.pallas_call(
        body,
        out_shape=jax.ShapeDtypeStruct((M_PER, N), jnp.float32),
        in_specs=[
            pl.BlockSpec(memory_space=pltpu.VMEM),
            pl.BlockSpec(memory_space=pltpu.VMEM),
        ],
        out_specs=pl.BlockSpec(memory_space=pltpu.VMEM),
        scratch_shapes=[
            pltpu.VMEM((N_DEV * ROWS, HC), bf16),
            pltpu.VMEM((N_DEV * ROWS, HC), bf16),
            pltpu.VMEM((4 * ROWS, HC), bf16),
            pltpu.VMEM((4 * ROWS, HC), bf16),
            pltpu.VMEM((2 * ROWS, HC), bf16),
            pltpu.VMEM((2 * ROWS, HC), bf16),
            pltpu.VMEM((1 * ROWS, HC), bf16),
            pltpu.VMEM((1 * ROWS, HC), bf16),
            pltpu.VMEM((1 * ROWS, HC), bf16),
            pltpu.VMEM((1 * ROWS, HC), bf16),
            pltpu.VMEM((2 * ROWS, HC), bf16),
            pltpu.VMEM((2 * ROWS, HC), bf16),
            pltpu.VMEM((4 * ROWS, HC), bf16),
            pltpu.VMEM((4 * ROWS, HC), bf16),
            pltpu.SemaphoreType.DMA((6,)),
            pltpu.SemaphoreType.DMA((6,)),
            pltpu.SemaphoreType.DMA((6,)),
            pltpu.SemaphoreType.DMA((6,)),
        ],
        compiler_params=pltpu.CompilerParams(
            collective_id=0,
            vmem_limit_bytes=64 * 1024 * 1024,
        ),
    )(t, W)


# device time: 122180 ns/iter; 1.8436x vs baseline; 1.0948x over previous
import jax
import jax.numpy as jnp
from jax import lax
from jax.experimental import pallas as pl
from jax.experimental.pallas import tpu as pltpu

N_DEV = 8
M_PER = 4096
ROWS = M_PER // N_DEV
K = 1024
N = 1024
HC = K // 2

RS_MASKS = ((1, 3, 4), (3, 4, 1))
AG_MASKS = ((4, 3, 1), (1, 4, 3))


def _sigma(bf: int, u: int) -> int:
    if bf == 0:
        a, b, c = (u >> 2) & 1, (u >> 1) & 1, u & 1
    else:
        b, c, a = (u >> 2) & 1, (u >> 1) & 1, u & 1
    return (a ^ b) | (b << 1) | (c << 2)


def kernel(t, W):
    def body(t_ref, w_ref, out_ref, accA, accB, w_bf,
             rsr0A, rsr0B, rsr1A, rsr1B, rsr2A, rsr2B,
             rs_ssem, rs_rsem, ag_ssem, ag_rsem):
        acc = (accA, accB)
        rsr = ((rsr0A, rsr1A, rsr2A), (rsr0B, rsr1B, rsr2B))

        me = lax.axis_index("i")
        p0, p1, p2 = me % 2, (me // 2) % 2, (me // 4) % 2
        f1, f3, f4 = p0 ^ p1, p1, p2
        vv = (4 * f1 + 2 * f3 + f4, 4 * f3 + 2 * f4 + f1)

        barrier_sem = pltpu.get_barrier_semaphore()
        for mask in (1, 3, 4):
            pl.semaphore_signal(barrier_sem, inc=1, device_id=(me ^ mask,),
                                device_id_type=pl.DeviceIdType.MESH)
        pl.semaphore_wait(barrier_sem, 3)

        def init_chunks(bf, want_sendhalf):
            halfbit0 = (vv[bf] // 4) % 2
            for u in range(N_DEV):
                cond = halfbit0 != ((u >> 2) & 1) if want_sendhalf \
                    else halfbit0 == ((u >> 2) & 1)
                @pl.when(cond)
                def _():
                    acc[bf][pl.ds(u * ROWS, ROWS), :] = t_ref[
                        pl.ds(_sigma(bf, u) * ROWS, ROWS),
                        pl.ds(bf * HC, HC)].astype(jnp.bfloat16)

        for bf in range(2):
            init_chunks(bf, want_sendhalf=True)

        step_waits = []
        for s in range(3):
            half = 4 >> s
            for bf in range(2):
                v = vv[bf]
                base = (v // (8 >> s)) * (8 >> s)
                halfbit = (v // half) % 2
                send_base = base + (1 - halfbit) * half
                keep_base = base + halfbit * half
                rdma = pltpu.make_async_remote_copy(
                    src_ref=acc[bf].at[pl.ds(send_base * ROWS, half * ROWS), :],
                    dst_ref=rsr[bf][s],
                    send_sem=rs_ssem.at[2 * s + bf],
                    recv_sem=rs_rsem.at[2 * s + bf],
                    device_id=(me ^ RS_MASKS[bf][s],),
                    device_id_type=pl.DeviceIdType.MESH,
                )
                rdma.start()
                step_waits.append((rdma, bf, keep_base, half, s))
            if s == 0:
                for bf in range(2):
                    init_chunks(bf, want_sendhalf=False)
                w_bf[:, :] = w_ref[:, :].astype(jnp.bfloat16)
            for rdma, bf, keep_base, half, ss in step_waits:
                rdma.wait()
                sl = pl.ds(keep_base * ROWS, half * ROWS)
                acc[bf][sl, :] = acc[bf][sl, :] + rsr[bf][ss][:, :]
            step_waits = []

        own = jnp.concatenate(
            [acc[0][pl.ds(vv[0] * ROWS, ROWS), :],
             acc[1][pl.ds(vv[1] * ROWS, ROWS), :]], axis=1)
        result = jnp.dot(own, w_bf[:, :], preferred_element_type=jnp.float32)
        rb = result.astype(jnp.bfloat16)
        acc[0][pl.ds(vv[0] * ROWS, ROWS), :] = rb[:, :HC]
        acc[1][pl.ds(vv[1] * ROWS, ROWS), :] = rb[:, HC:]

        for s in range(3):
            bsz = 1 << s
            waits = []
            for bf in range(2):
                v = vv[bf]
                b_s = (v // bsz) * bsz
                rdma = pltpu.make_async_remote_copy(
                    src_ref=acc[bf].at[pl.ds(b_s * ROWS, bsz * ROWS), :],
                    dst_ref=acc[bf].at[pl.ds(b_s * ROWS, bsz * ROWS), :],
                    send_sem=ag_ssem.at[2 * s + bf],
                    recv_sem=ag_rsem.at[2 * s + bf],
                    device_id=(me ^ AG_MASKS[bf][s],),
                    device_id_type=pl.DeviceIdType.MESH,
                )
                rdma.start()
                waits.append(rdma)
            for rdma in waits:
                rdma.wait()

        for bf in range(2):
            for u in range(N_DEV):
                out_ref[pl.ds(_sigma(bf, u) * ROWS, ROWS),
                        pl.ds(bf * HC, HC)] = (
                    acc[bf][pl.ds(u * ROWS, ROWS), :].astype(jnp.float32))

    bf16 = jnp.bfloat16
    return pl.pallas_call(
        body,
        out_shape=jax.ShapeDtypeStruct((M_PER, N), jnp.float32),
        in_specs=[
            pl.BlockSpec(memory_space=pltpu.VMEM),
            pl.BlockSpec(memory_space=pltpu.VMEM),
        ],
        out_specs=pl.BlockSpec(memory_space=pltpu.VMEM),
        scratch_shapes=[
            pltpu.VMEM((N_DEV * ROWS, HC), bf16),
            pltpu.VMEM((N_DEV * ROWS, HC), bf16),
            pltpu.VMEM((K, N), bf16),
            pltpu.VMEM((4 * ROWS, HC), bf16),
            pltpu.VMEM((4 * ROWS, HC), bf16),
            pltpu.VMEM((2 * ROWS, HC), bf16),
            pltpu.VMEM((2 * ROWS, HC), bf16),
            pltpu.VMEM((1 * ROWS, HC), bf16),
            pltpu.VMEM((1 * ROWS, HC), bf16),
            pltpu.SemaphoreType.DMA((6,)),
            pltpu.SemaphoreType.DMA((6,)),
            pltpu.SemaphoreType.DMA((6,)),
            pltpu.SemaphoreType.DMA((6,)),
        ],
        compiler_params=pltpu.CompilerParams(
            collective_id=0,
            vmem_limit_bytes=64 * 1024 * 1024,
        ),
    )(t, W)


# device time: 102265 ns/iter; 2.2026x vs baseline; 1.1947x over previous
import jax
import jax.numpy as jnp
from jax import lax
from jax.experimental import pallas as pl
from jax.experimental.pallas import tpu as pltpu

N_DEV = 8
M_PER = 4096
ROWS = M_PER // N_DEV
K = 1024
N = 1024
HC = K // 2

RS_MASKS = ((1, 3, 4), (4, 3, 1))


def _fbits(bf: int, u):
    b2, b1, b0 = (u // 4) % 2, (u // 2) % 2, u % 2
    return (b2, b1, b0) if bf == 0 else (b0, b1, b2)


def _sigma(bf: int, u):
    f1, f3, f4 = _fbits(bf, u)
    return (f1 ^ f3) + 2 * f3 + 4 * f4


def kernel(t, W):
    def body(t_ref, w_ref, out_ref, accA, accB, w_bf,
             rsr0A, rsr0B, rsr1A, rsr1B, rsr2A, rsr2B,
             rs_ssem, rs_rsem, ag_ssem, ag_rsem):
        acc = (accA, accB)
        rsr = ((rsr0A, rsr1A, rsr2A), (rsr0B, rsr1B, rsr2B))

        me = lax.axis_index("i")
        p0, p1, p2 = me % 2, (me // 2) % 2, (me // 4) % 2
        f1, f3, f4 = p0 ^ p1, p1, p2
        vv = (4 * f1 + 2 * f3 + f4, 4 * f4 + 2 * f3 + f1)

        def rows(ref, start_chunk, n_chunks):
            return ref.at[pl.ds(start_chunk * ROWS, n_chunks * ROWS), :]

        def rsend(bf, slot, src, dst, n_chunks, mask):
            return pltpu.make_async_remote_copy(
                src_ref=src, dst_ref=dst,
                send_sem=rs_ssem.at[5 * bf + slot],
                recv_sem=rs_rsem.at[5 * bf + slot],
                device_id=(me ^ mask,),
                device_id_type=pl.DeviceIdType.MESH,
            )

        barrier_sem = pltpu.get_barrier_semaphore()
        for mask in (1, 3, 4):
            pl.semaphore_signal(barrier_sem, inc=1, device_id=(me ^ mask,),
                                device_id_type=pl.DeviceIdType.MESH)
        pl.semaphore_wait(barrier_sem, 3)

        hb1, hb2, hb3 = [], [], []
        keep0, keep1, sb0, sb1, sb2 = [], [], [], [], []
        for bf in range(2):
            v = vv[bf]
            h1, h2, h3 = (v // 4) % 2, (v // 2) % 2, v % 2
            hb1.append(h1); hb2.append(h2); hb3.append(h3)
            keep0.append(h1 * 4)
            sb0.append((1 - h1) * 4)
            keep1.append(h1 * 4 + h2 * 2)
            sb1.append(h1 * 4 + (1 - h2) * 2)
            sb2.append(keep1[bf] + (1 - h3))

        def init_chunks(bf, want_sendhalf):
            for u in range(N_DEV):
                cond = (hb1[bf] != (u >> 2)) if want_sendhalf \
                    else (hb1[bf] == (u >> 2))
                @pl.when(cond)
                def _():
                    acc[bf][pl.ds(u * ROWS, ROWS), :] = t_ref[
                        pl.ds(_sigma(bf, u) * ROWS, ROWS),
                        pl.ds(bf * HC, HC)].astype(jnp.bfloat16)

        s0sub1, s0sub2, s1sub1, s1sub2, s2x = [], [], [], [], []

        for bf in range(2):
            init_chunks(bf, want_sendhalf=True)
        for bf in range(2):
            m = RS_MASKS[bf][0]
            r = rsend(bf, 0, rows(acc[bf], sb0[bf] + (1 - hb2[bf]) * 2, 2),
                      rows(rsr[bf][0], 0, 2), 2, m)
            r.start()
            s0sub1.append(r)

        for bf in range(2):
            init_chunks(bf, want_sendhalf=False)
        w_bf[:, :] = w_ref[:, :].astype(jnp.bfloat16)

        for bf in range(2):
            s0sub1[bf].wait()
            sl = pl.ds(sb1[bf] * ROWS, 2 * ROWS)
            acc[bf][sl, :] = acc[bf][sl, :] + rsr[bf][0][pl.ds(0, 2 * ROWS), :]
        for bf in range(2):
            r = rsend(bf, 2, rows(acc[bf], sb1[bf] + (1 - hb3[bf]), 1),
                      rows(rsr[bf][1], 0, 1), 1, RS_MASKS[bf][1])
            r.start()
            s1sub1.append(r)
            r = rsend(bf, 1, rows(acc[bf], sb0[bf] + hb2[bf] * 2, 2),
                      rows(rsr[bf][0], 2, 2), 2, RS_MASKS[bf][0])
            r.start()
            s0sub2.append(r)

        for bf in range(2):
            s0sub2[bf].wait()
            sl = pl.ds(keep1[bf] * ROWS, 2 * ROWS)
            acc[bf][sl, :] = acc[bf][sl, :] + rsr[bf][0][pl.ds(2 * ROWS, 2 * ROWS), :]
            s1sub1[bf].wait()
            sl = pl.ds(sb2[bf] * ROWS, ROWS)
            acc[bf][sl, :] = acc[bf][sl, :] + rsr[bf][1][pl.ds(0, ROWS), :]
        for bf in range(2):
            r = rsend(bf, 4, rows(acc[bf], sb2[bf], 1),
                      rows(rsr[bf][2], 0, 1), 1, RS_MASKS[bf][2])
            r.start()
            s2x.append(r)
            r = rsend(bf, 3, rows(acc[bf], sb1[bf] + hb3[bf], 1),
                      rows(rsr[bf][1], 1, 1), 1, RS_MASKS[bf][1])
            r.start()
            s1sub2.append(r)

        for bf in range(2):
            v = vv[bf]
            s1sub2[bf].wait()
            sl = pl.ds(v * ROWS, ROWS)
            acc[bf][sl, :] = acc[bf][sl, :] + rsr[bf][1][pl.ds(ROWS, ROWS), :]
            s2x[bf].wait()
            acc[bf][sl, :] = acc[bf][sl, :] + rsr[bf][2][pl.ds(0, ROWS), :]

        own = jnp.concatenate(
            [acc[0][pl.ds(vv[0] * ROWS, ROWS), :],
             acc[1][pl.ds(vv[1] * ROWS, ROWS), :]], axis=1)

        def agsend(bf, slot, u_src, mask):
            return pltpu.make_async_remote_copy(
                src_ref=rows(acc[bf], u_src, 1),
                dst_ref=rows(acc[bf], u_src, 1),
                send_sem=ag_ssem.at[7 * bf + slot],
                recv_sem=ag_rsem.at[7 * bf + slot],
                device_id=(me ^ mask,),
                device_id_type=pl.DeviceIdType.MESH,
            )

        def store_out(bf, u):
            out_ref[pl.ds(_sigma(bf, u) * ROWS, ROWS),
                    pl.ds(bf * HC, HC)] = (
                acc[bf][pl.ds(u * ROWS, ROWS), :].astype(jnp.float32))

        S = [[None] * 7 for _ in range(2)]
        for bf in range(2):
            m0, m1, m2 = RS_MASKS[bf][::-1]
            half = jnp.dot(own, w_bf[:, pl.ds(bf * HC, HC)],
                           preferred_element_type=jnp.float32)
            out_ref[pl.ds(me * ROWS, ROWS), pl.ds(bf * HC, HC)] = half
            v = vv[bf]
            acc[bf][pl.ds(v * ROWS, ROWS), :] = half.astype(jnp.bfloat16)
            for slot, mask in ((0, m0), (1, m1), (2, m2)):
                S[bf][slot] = agsend(bf, slot, v, mask)
                S[bf][slot].start()

        for bf in range(2):
            m0, m1, m2 = RS_MASKS[bf][::-1]
            v = vv[bf]
            S[bf][0].wait_recv()
            for slot, mask, u in ((3, m1, v ^ 1), (4, m2, v ^ 1)):
                S[bf][slot] = agsend(bf, slot, u, mask)
                S[bf][slot].start()
            store_out(bf, v ^ 1)
        for bf in range(2):
            m0, m1, m2 = RS_MASKS[bf][::-1]
            v = vv[bf]
            S[bf][1].wait_recv()
            S[bf][5] = agsend(bf, 5, v ^ 2, m2)
            S[bf][5].start()
            store_out(bf, v ^ 2)
        for bf in range(2):
            m0, m1, m2 = RS_MASKS[bf][::-1]
            v = vv[bf]
            S[bf][3].wait_recv()
            S[bf][6] = agsend(bf, 6, v ^ 3, m2)
            S[bf][6].start()
            store_out(bf, v ^ 3)
        for slot, off in ((2, 4), (4, 5), (5, 6), (6, 7)):
            for bf in range(2):
                S[bf][slot].wait_recv()
                store_out(bf, vv[bf] ^ off)
        for bf in range(2):
            for slot in range(7):
                S[bf][slot].wait_send()

    bf16 = jnp.bfloat16
    return pl.pallas_call(
        body,
        out_shape=jax.ShapeDtypeStruct((M_PER, N), jnp.float32),
        in_specs=[
            pl.BlockSpec(memory_space=pltpu.VMEM),
            pl.BlockSpec(memory_space=pltpu.VMEM),
        ],
        out_specs=pl.BlockSpec(memory_space=pltpu.VMEM),
        scratch_shapes=[
            pltpu.VMEM((N_DEV * ROWS, HC), bf16),
            pltpu.VMEM((N_DEV * ROWS, HC), bf16),
            pltpu.VMEM((K, N), bf16),
            pltpu.VMEM((4 * ROWS, HC), bf16),
            pltpu.VMEM((4 * ROWS, HC), bf16),
            pltpu.VMEM((2 * ROWS, HC), bf16),
            pltpu.VMEM((2 * ROWS, HC), bf16),
            pltpu.VMEM((1 * ROWS, HC), bf16),
            pltpu.VMEM((1 * ROWS, HC), bf16),
            pltpu.SemaphoreType.DMA((10,)),
            pltpu.SemaphoreType.DMA((10,)),
            pltpu.SemaphoreType.DMA((14,)),
            pltpu.SemaphoreType.DMA((14,)),
        ],
        compiler_params=pltpu.CompilerParams(
            collective_id=0,
            vmem_limit_bytes=64 * 1024 * 1024,
        ),
    )(t, W)


# device time: 96214 ns/iter; 2.3412x vs baseline; 1.0629x over previous
import jax
import jax.numpy as jnp
from jax import lax
from jax.experimental import pallas as pl
from jax.experimental.pallas import tpu as pltpu

N_DEV = 8
M_PER = 4096
ROWS = M_PER // N_DEV
K = 1024
N = 1024
HC = K // 2

RS_MASKS = ((1, 3, 4), (4, 3, 1))


def _fbits(bf: int, u):
    b2, b1, b0 = (u // 4) % 2, (u // 2) % 2, u % 2
    return (b2, b1, b0) if bf == 0 else (b0, b1, b2)


def _sigma(bf: int, u):
    f1, f3, f4 = _fbits(bf, u)
    return (f1 ^ f3) + 2 * f3 + 4 * f4


def kernel(t, W):
    def body(t_hbm, w_hbm, out_hbm, accA, accB, t_vm, w_vm, w_bf, stg,
             rsr0A, rsr0B, rsr1A, rsr1B, rsr2A, rsr2B,
             rs_ssem, rs_rsem, ag_ssem, ag_rsem, in_sems, out_sems):
        acc = (accA, accB)
        rsr = ((rsr0A, rsr1A, rsr2A), (rsr0B, rsr1B, rsr2B))
        t_ref, w_ref = t_vm, w_vm

        t_dma = pltpu.make_async_copy(t_hbm, t_vm, in_sems.at[0])
        w_dma = pltpu.make_async_copy(w_hbm, w_vm, in_sems.at[1])
        t_dma.start()
        w_dma.start()

        me = lax.axis_index("i")
        p0, p1, p2 = me % 2, (me // 2) % 2, (me // 4) % 2
        f1, f3, f4 = p0 ^ p1, p1, p2
        vv = (4 * f1 + 2 * f3 + f4, 4 * f4 + 2 * f3 + f1)

        def rows(ref, start_chunk, n_chunks):
            return ref.at[pl.ds(start_chunk * ROWS, n_chunks * ROWS), :]

        def rsend(bf, slot, src, dst, n_chunks, mask):
            return pltpu.make_async_remote_copy(
                src_ref=src, dst_ref=dst,
                send_sem=rs_ssem.at[5 * bf + slot],
                recv_sem=rs_rsem.at[5 * bf + slot],
                device_id=(me ^ mask,),
                device_id_type=pl.DeviceIdType.MESH,
            )

        barrier_sem = pltpu.get_barrier_semaphore()
        for mask in (1, 3, 4):
            pl.semaphore_signal(barrier_sem, inc=1, device_id=(me ^ mask,),
                                device_id_type=pl.DeviceIdType.MESH)
        pl.semaphore_wait(barrier_sem, 3)
        t_dma.wait()

        hb1, hb2, hb3 = [], [], []
        keep0, keep1, sb0, sb1, sb2 = [], [], [], [], []
        for bf in range(2):
            v = vv[bf]
            h1, h2, h3 = (v // 4) % 2, (v // 2) % 2, v % 2
            hb1.append(h1); hb2.append(h2); hb3.append(h3)
            keep0.append(h1 * 4)
            sb0.append((1 - h1) * 4)
            keep1.append(h1 * 4 + h2 * 2)
            sb1.append(h1 * 4 + (1 - h2) * 2)
            sb2.append(keep1[bf] + (1 - h3))

        def init_chunks(bf, want_sendhalf):
            for u in range(N_DEV):
                cond = (hb1[bf] != (u >> 2)) if want_sendhalf \
                    else (hb1[bf] == (u >> 2))
                @pl.when(cond)
                def _():
                    acc[bf][pl.ds(u * ROWS, ROWS), :] = t_ref[
                        pl.ds(_sigma(bf, u) * ROWS, ROWS),
                        pl.ds(bf * HC, HC)].astype(jnp.bfloat16)

        s0sub1, s0sub2, s1sub1, s1sub2, s2x = [], [], [], [], []

        for bf in range(2):
            init_chunks(bf, want_sendhalf=True)
        for bf in range(2):
            m = RS_MASKS[bf][0]
            r = rsend(bf, 0, rows(acc[bf], sb0[bf] + (1 - hb2[bf]) * 2, 2),
                      rows(rsr[bf][0], 0, 2), 2, m)
            r.start()
            s0sub1.append(r)

        for bf in range(2):
            init_chunks(bf, want_sendhalf=False)
        w_dma.wait()
        w_bf[:, :] = w_ref[:, :].astype(jnp.bfloat16)

        for bf in range(2):
            s0sub1[bf].wait()
            sl = pl.ds(sb1[bf] * ROWS, 2 * ROWS)
            acc[bf][sl, :] = acc[bf][sl, :] + rsr[bf][0][pl.ds(0, 2 * ROWS), :]
        for bf in range(2):
            r = rsend(bf, 2, rows(acc[bf], sb1[bf] + (1 - hb3[bf]), 1),
                      rows(rsr[bf][1], 0, 1), 1, RS_MASKS[bf][1])
            r.start()
            s1sub1.append(r)
            r = rsend(bf, 1, rows(acc[bf], sb0[bf] + hb2[bf] * 2, 2),
                      rows(rsr[bf][0], 2, 2), 2, RS_MASKS[bf][0])
            r.start()
            s0sub2.append(r)

        for bf in range(2):
            s0sub2[bf].wait()
            sl = pl.ds(keep1[bf] * ROWS, 2 * ROWS)
            acc[bf][sl, :] = acc[bf][sl, :] + rsr[bf][0][pl.ds(2 * ROWS, 2 * ROWS), :]
            s1sub1[bf].wait()
            sl = pl.ds(sb2[bf] * ROWS, ROWS)
            acc[bf][sl, :] = acc[bf][sl, :] + rsr[bf][1][pl.ds(0, ROWS), :]
        for bf in range(2):
            r = rsend(bf, 4, rows(acc[bf], sb2[bf], 1),
                      rows(rsr[bf][2], 0, 1), 1, RS_MASKS[bf][2])
            r.start()
            s2x.append(r)
            r = rsend(bf, 3, rows(acc[bf], sb1[bf] + hb3[bf], 1),
                      rows(rsr[bf][1], 1, 1), 1, RS_MASKS[bf][1])
            r.start()
            s1sub2.append(r)

        for bf in range(2):
            v = vv[bf]
            s1sub2[bf].wait()
            sl = pl.ds(v * ROWS, ROWS)
            acc[bf][sl, :] = acc[bf][sl, :] + rsr[bf][1][pl.ds(ROWS, ROWS), :]
            s2x[bf].wait()
            acc[bf][sl, :] = acc[bf][sl, :] + rsr[bf][2][pl.ds(0, ROWS), :]

        own = jnp.concatenate(
            [acc[0][pl.ds(vv[0] * ROWS, ROWS), :],
             acc[1][pl.ds(vv[1] * ROWS, ROWS), :]], axis=1)

        def agsend(bf, slot, u_src, mask):
            return pltpu.make_async_remote_copy(
                src_ref=rows(acc[bf], u_src, 1),
                dst_ref=rows(acc[bf], u_src, 1),
                send_sem=ag_ssem.at[7 * bf + slot],
                recv_sem=ag_rsem.at[7 * bf + slot],
                device_id=(me ^ mask,),
                device_id_type=pl.DeviceIdType.MESH,
            )

        store_dmas = []

        def store_f32(bf, row_chunk, val):
            j = len(store_dmas)
            slot = j % 4
            if j >= 4:
                store_dmas[j - 4].wait()
            stg[slot] = val
            dma = pltpu.make_async_copy(
                stg.at[slot],
                out_hbm.at[pl.ds(row_chunk * ROWS, ROWS),
                           pl.ds(bf * HC, HC)],
                out_sems.at[j])
            dma.start()
            store_dmas.append(dma)

        def store_out(bf, u):
            store_f32(bf, _sigma(bf, u),
                      acc[bf][pl.ds(u * ROWS, ROWS), :].astype(jnp.float32))

        S = [[None] * 7 for _ in range(2)]
        for bf in range(2):
            m0, m1, m2 = RS_MASKS[bf][::-1]
            half = jnp.dot(own, w_bf[:, pl.ds(bf * HC, HC)],
                           preferred_element_type=jnp.float32)
            store_f32(bf, me, half)
            v = vv[bf]
            acc[bf][pl.ds(v * ROWS, ROWS), :] = half.astype(jnp.bfloat16)
            for slot, mask in ((0, m0), (1, m1), (2, m2)):
                S[bf][slot] = agsend(bf, slot, v, mask)
                S[bf][slot].start()

        for bf in range(2):
            m0, m1, m2 = RS_MASKS[bf][::-1]
            v = vv[bf]
            S[bf][0].wait_recv()
            for slot, mask, u in ((3, m1, v ^ 1), (4, m2, v ^ 1)):
                S[bf][slot] = agsend(bf, slot, u, mask)
                S[bf][slot].start()
            store_out(bf, v ^ 1)
        for bf in range(2):
            m0, m1, m2 = RS_MASKS[bf][::-1]
            v = vv[bf]
            S[bf][1].wait_recv()
            S[bf][5] = agsend(bf, 5, v ^ 2, m2)
            S[bf][5].start()
            store_out(bf, v ^ 2)
        for bf in range(2):
            m0, m1, m2 = RS_MASKS[bf][::-1]
            v = vv[bf]
            S[bf][3].wait_recv()
            S[bf][6] = agsend(bf, 6, v ^ 3, m2)
            S[bf][6].start()
            store_out(bf, v ^ 3)
        for slot, off in ((2, 4), (4, 5), (5, 6), (6, 7)):
            for bf in range(2):
                S[bf][slot].wait_recv()
                store_out(bf, vv[bf] ^ off)
        for bf in range(2):
            for slot in range(7):
                S[bf][slot].wait_send()
        for dma in store_dmas[-4:]:
            dma.wait()

    bf16 = jnp.bfloat16
    return pl.pallas_call(
        body,
        out_shape=jax.ShapeDtypeStruct((M_PER, N), jnp.float32),
        in_specs=[
            pl.BlockSpec(memory_space=pltpu.MemorySpace.HBM),
            pl.BlockSpec(memory_space=pltpu.MemorySpace.HBM),
        ],
        out_specs=pl.BlockSpec(memory_space=pltpu.MemorySpace.HBM),
        scratch_shapes=[
            pltpu.VMEM((N_DEV * ROWS, HC), bf16),
            pltpu.VMEM((N_DEV * ROWS, HC), bf16),
            pltpu.VMEM((M_PER, K), jnp.float32),
            pltpu.VMEM((K, N), jnp.float32),
            pltpu.VMEM((K, N), bf16),
            pltpu.VMEM((4, ROWS, HC), jnp.float32),
            pltpu.VMEM((4 * ROWS, HC), bf16),
            pltpu.VMEM((4 * ROWS, HC), bf16),
            pltpu.VMEM((2 * ROWS, HC), bf16),
            pltpu.VMEM((2 * ROWS, HC), bf16),
            pltpu.VMEM((1 * ROWS, HC), bf16),
            pltpu.VMEM((1 * ROWS, HC), bf16),
            pltpu.SemaphoreType.DMA((10,)),
            pltpu.SemaphoreType.DMA((10,)),
            pltpu.SemaphoreType.DMA((14,)),
            pltpu.SemaphoreType.DMA((14,)),
            pltpu.SemaphoreType.DMA((2,)),
            pltpu.SemaphoreType.DMA((16,)),
        ],
        compiler_params=pltpu.CompilerParams(
            collective_id=0,
            vmem_limit_bytes=64 * 1024 * 1024,
        ),
    )(t, W)


# device time: 88045 ns/iter; 2.5584x vs baseline; 1.0928x over previous
import jax
import jax.numpy as jnp
from jax import lax
from jax.experimental import pallas as pl
from jax.experimental.pallas import tpu as pltpu

N_DEV = 8
M_PER = 4096
ROWS = M_PER // N_DEV
HR = ROWS // 2
K = 1024
N = 1024
HC = K // 2

RS_MASKS = ((1, 3, 4), (4, 3, 1))


def _fbits(bf: int, u):
    b2, b1, b0 = (u // 4) % 2, (u // 2) % 2, u % 2
    return (b2, b1, b0) if bf == 0 else (b0, b1, b2)


def _sigma(bf: int, u):
    f1, f3, f4 = _fbits(bf, u)
    return (f1 ^ f3) + 2 * f3 + 4 * f4


def kernel(t, W):
    def body(t_hbm, w_hbm, out_hbm,
             accA0, accB0, accA1, accB1, t_vm, w_vm, w_bf, stg,
             r0A0, r0B0, r1A0, r1B0, r2A0, r2B0,
             r0A1, r0B1, r1A1, r1B1, r2A1, r2B1,
             rs_ssem, rs_rsem, ag_ssem, ag_rsem, in_sems, out_sems):
        acc = ((accA0, accB0), (accA1, accB1))
        rsr = (((r0A0, r1A0, r2A0), (r0B0, r1B0, r2B0)),
               ((r0A1, r1A1, r2A1), (r0B1, r1B1, r2B1)))

        t_dmas = [[None] * N_DEV for _ in range(2)]
        for st in range(2):
            for c in range(N_DEV):
                sl = pl.ds(c * ROWS + st * HR, HR)
                t_dmas[st][c] = pltpu.make_async_copy(
                    t_hbm.at[sl, :], t_vm.at[sl, :],
                    in_sems.at[1 + N_DEV * st + c])
                t_dmas[st][c].start()
        w_dma = pltpu.make_async_copy(w_hbm, w_vm, in_sems.at[0])
        w_dma.start()

        me = lax.axis_index("i")
        p0, p1, p2 = me % 2, (me // 2) % 2, (me // 4) % 2
        f1, f3, f4 = p0 ^ p1, p1, p2
        vv = (4 * f1 + 2 * f3 + f4, 4 * f4 + 2 * f3 + f1)

        def rows(ref, start_chunk, n_chunks):
            return ref.at[pl.ds(start_chunk * HR, n_chunks * HR), :]

        barrier_sem = pltpu.get_barrier_semaphore()
        for mask in (1, 3, 4):
            pl.semaphore_signal(barrier_sem, inc=1, device_id=(me ^ mask,),
                                device_id_type=pl.DeviceIdType.MESH)

        hb1, hb2, hb3 = [], [], []
        keep1, sb0, sb1, sb2 = [], [], [], []
        for bf in range(2):
            v = vv[bf]
            h1, h2, h3 = (v // 4) % 2, (v // 2) % 2, v % 2
            hb1.append(h1); hb2.append(h2); hb3.append(h3)
            sb0.append((1 - h1) * 4)
            keep1.append(h1 * 4 + h2 * 2)
            sb1.append(h1 * 4 + (1 - h2) * 2)
            sb2.append(keep1[bf] + (1 - h3))

        def rsend(st, bf, slot, src, dst, mask):
            return pltpu.make_async_remote_copy(
                src_ref=src, dst_ref=dst,
                send_sem=rs_ssem.at[10 * st + 5 * bf + slot],
                recv_sem=rs_rsem.at[10 * st + 5 * bf + slot],
                device_id=(me ^ mask,),
                device_id_type=pl.DeviceIdType.MESH,
            )

        def init_chunks(st, bf, want_sendhalf):
            for u in range(N_DEV):
                cond = (hb1[bf] != (u >> 2)) if want_sendhalf \
                    else (hb1[bf] == (u >> 2))
                @pl.when(cond)
                def _():
                    acc[st][bf][pl.ds(u * HR, HR), :] = t_vm[
                        pl.ds(_sigma(bf, u) * ROWS + st * HR, HR),
                        pl.ds(bf * HC, HC)].astype(jnp.bfloat16)

        R = [{} for _ in range(2)]

        def rs_a(st):
            for c in range(N_DEV):
                t_dmas[st][c].wait()
            for bf in range(2):
                init_chunks(st, bf, want_sendhalf=True)
            if st == 0:
                pl.semaphore_wait(barrier_sem, 3)
            for bf in range(2):
                r = rsend(st, bf, 0,
                          rows(acc[st][bf], sb0[bf] + (1 - hb2[bf]) * 2, 2),
                          rows(rsr[st][bf][0], 0, 2), RS_MASKS[bf][0])
                r.start()
                R[st][('s0sub1', bf)] = r

        def rs_b(st):
            for bf in range(2):
                init_chunks(st, bf, want_sendhalf=False)

        def rs_c(st):
            for bf in range(2):
                R[st][('s0sub1', bf)].wait()
                sl = pl.ds(sb1[bf] * HR, 2 * HR)
                acc[st][bf][sl, :] = (acc[st][bf][sl, :]
                                      + rsr[st][bf][0][pl.ds(0, 2 * HR), :])
            for bf in range(2):
                r = rsend(st, bf, 2,
                          rows(acc[st][bf], sb1[bf] + (1 - hb3[bf]), 1),
                          rows(rsr[st][bf][1], 0, 1), RS_MASKS[bf][1])
                r.start()
                R[st][('s1sub1', bf)] = r
                r = rsend(st, bf, 1,
                          rows(acc[st][bf], sb0[bf] + hb2[bf] * 2, 2),
                          rows(rsr[st][bf][0], 2, 2), RS_MASKS[bf][0])
                r.start()
                R[st][('s0sub2', bf)] = r

        def rs_d(st):
            for bf in range(2):
                R[st][('s0sub2', bf)].wait()
                sl = pl.ds(keep1[bf] * HR, 2 * HR)
                acc[st][bf][sl, :] = (acc[st][bf][sl, :]
                                      + rsr[st][bf][0][pl.ds(2 * HR, 2 * HR), :])
                R[st][('s1sub1', bf)].wait()
                sl = pl.ds(sb2[bf] * HR, HR)
                acc[st][bf][sl, :] = (acc[st][bf][sl, :]
                                      + rsr[st][bf][1][pl.ds(0, HR), :])
            for bf in range(2):
                r = rsend(st, bf, 4, rows(acc[st][bf], sb2[bf], 1),
                          rows(rsr[st][bf][2], 0, 1), RS_MASKS[bf][2])
                r.start()
                R[st][('s2', bf)] = r
                r = rsend(st, bf, 3,
                          rows(acc[st][bf], sb1[bf] + hb3[bf], 1),
                          rows(rsr[st][bf][1], 1, 1), RS_MASKS[bf][1])
                r.start()
                R[st][('s1sub2', bf)] = r

        def rs_e(st):
            for bf in range(2):
                v = vv[bf]
                R[st][('s1sub2', bf)].wait()
                sl = pl.ds(v * HR, HR)
                acc[st][bf][sl, :] = (acc[st][bf][sl, :]
                                      + rsr[st][bf][1][pl.ds(HR, HR), :])
                R[st][('s2', bf)].wait()
                acc[st][bf][sl, :] = (acc[st][bf][sl, :]
                                      + rsr[st][bf][2][pl.ds(0, HR), :])

        store_dmas = []

        def store_f32(st, bf, row_chunk, val):
            j = len(store_dmas)
            slot = j % 4
            if j >= 4:
                store_dmas[j - 4].wait()
            stg[slot] = val
            dma = pltpu.make_async_copy(
                stg.at[slot],
                out_hbm.at[pl.ds(row_chunk * ROWS + st * HR, HR),
                           pl.ds(bf * HC, HC)],
                out_sems.at[j])
            dma.start()
            store_dmas.append(dma)

        def store_out(st, bf, u):
            store_f32(st, bf, _sigma(bf, u),
                      acc[st][bf][pl.ds(u * HR, HR), :].astype(jnp.float32))

        S = [[[None] * 7 for _ in range(2)] for _ in range(2)]

        def agsend(st, bf, slot, u_src, mask):
            return pltpu.make_async_remote_copy(
                src_ref=rows(acc[st][bf], u_src, 1),
                dst_ref=rows(acc[st][bf], u_src, 1),
                send_sem=ag_ssem.at[14 * st + 7 * bf + slot],
                recv_sem=ag_rsem.at[14 * st + 7 * bf + slot],
                device_id=(me ^ mask,),
                device_id_type=pl.DeviceIdType.MESH,
            )

        def mm_ag0(st):
            own = jnp.concatenate(
                [acc[st][0][pl.ds(vv[0] * HR, HR), :],
                 acc[st][1][pl.ds(vv[1] * HR, HR), :]], axis=1)
            for bf in range(2):
                m0, m1, m2 = RS_MASKS[bf][::-1]
                half = jnp.dot(own, w_bf[:, pl.ds(bf * HC, HC)],
                               preferred_element_type=jnp.float32)
                store_f32(st, bf, me, half)
                v = vv[bf]
                acc[st][bf][pl.ds(v * HR, HR), :] = half.astype(jnp.bfloat16)
                for slot, mask in ((0, m0), (1, m1), (2, m2)):
                    S[st][bf][slot] = agsend(st, bf, slot, v, mask)
                    S[st][bf][slot].start()

        def ag1(st):
            for bf in range(2):
                m0, m1, m2 = RS_MASKS[bf][::-1]
                v = vv[bf]
                S[st][bf][0].wait_recv()
                for slot, mask in ((3, m1), (4, m2)):
                    S[st][bf][slot] = agsend(st, bf, slot, v ^ 1, mask)
                    S[st][bf][slot].start()
                store_out(st, bf, v ^ 1)

        def ag2(st):
            for bf in range(2):
                m2 = RS_MASKS[bf][0]
                v = vv[bf]
                S[st][bf][1].wait_recv()
                S[st][bf][5] = agsend(st, bf, 5, v ^ 2, m2)
                S[st][bf][5].start()
                store_out(st, bf, v ^ 2)
            for bf in range(2):
                m2 = RS_MASKS[bf][0]
                v = vv[bf]
                S[st][bf][3].wait_recv()
                S[st][bf][6] = agsend(st, bf, 6, v ^ 3, m2)
                S[st][bf][6].start()
                store_out(st, bf, v ^ 3)

        def ag3(st):
            for slot, off in ((2, 4), (4, 5), (5, 6), (6, 7)):
                for bf in range(2):
                    S[st][bf][slot].wait_recv()
                    store_out(st, bf, vv[bf] ^ off)

        def ag_drain(st):
            for bf in range(2):
                for slot in range(7):
                    S[st][bf][slot].wait_send()

        rs_a(0)
        rs_b(0)
        w_dma.wait()
        w_bf[:, :] = w_vm[:, :].astype(jnp.bfloat16)
        rs_c(0)
        rs_d(0)
        rs_a(1)
        rs_e(0)
        mm_ag0(0)
        rs_b(1)
        rs_c(1)
        ag1(0)
        rs_d(1)
        ag2(0)
        rs_e(1)
        ag3(0)
        mm_ag0(1)
        ag1(1)
        ag2(1)
        ag3(1)
        ag_drain(0)
        ag_drain(1)
        for dma in store_dmas[-4:]:
            dma.wait()

    bf16 = jnp.bfloat16
    f32 = jnp.float32
    return pl.pallas_call(
        body,
        out_shape=jax.ShapeDtypeStruct((M_PER, N), f32),
        in_specs=[
            pl.BlockSpec(memory_space=pltpu.MemorySpace.HBM),
            pl.BlockSpec(memory_space=pltpu.MemorySpace.HBM),
        ],
        out_specs=pl.BlockSpec(memory_space=pltpu.MemorySpace.HBM),
        scratch_shapes=[
            pltpu.VMEM((N_DEV * HR, HC), bf16),
            pltpu.VMEM((N_DEV * HR, HC), bf16),
            pltpu.VMEM((N_DEV * HR, HC), bf16),
            pltpu.VMEM((N_DEV * HR, HC), bf16),
            pltpu.VMEM((M_PER, K), f32),
            pltpu.VMEM((K, N), f32),
            pltpu.VMEM((K, N), bf16),
            pltpu.VMEM((4, HR, HC), f32),
            pltpu.VMEM((4 * HR, HC), bf16),
            pltpu.VMEM((4 * HR, HC), bf16),
            pltpu.VMEM((2 * HR, HC), bf16),
            pltpu.VMEM((2 * HR, HC), bf16),
            pltpu.VMEM((1 * HR, HC), bf16),
            pltpu.VMEM((1 * HR, HC), bf16),
            pltpu.VMEM((4 * HR, HC), bf16),
            pltpu.VMEM((4 * HR, HC), bf16),
            pltpu.VMEM((2 * HR, HC), bf16),
            pltpu.VMEM((2 * HR, HC), bf16),
            pltpu.VMEM((1 * HR, HC), bf16),
            pltpu.VMEM((1 * HR, HC), bf16),
            pltpu.SemaphoreType.DMA((20,)),
            pltpu.SemaphoreType.DMA((20,)),
            pltpu.SemaphoreType.DMA((28,)),
            pltpu.SemaphoreType.DMA((28,)),
            pltpu.SemaphoreType.DMA((17,)),
            pltpu.SemaphoreType.DMA((32,)),
        ],
        compiler_params=pltpu.CompilerParams(
            collective_id=0,
            vmem_limit_bytes=64 * 1024 * 1024,
        ),
    )(t, W)
